# Initial kernel scaffold; baseline (speedup 1.0000x reference)
import jax
import jax.numpy as jnp
from jax.experimental import pallas as pl


def kernel(x, t, class_labels, w, bias, tproj):
    raise NotImplementedError("write your pallas kernel here")


# speedup vs baseline: 2.1761x; 2.1761x over previous
import jax
import jax.numpy as jnp
from jax.experimental import pallas as pl
from jax.experimental.pallas import tpu as pltpu


def _cond_conv_kernel(x_ref,
                      wx_ref,
                      wctx_ref,
                      lab_ref,
                      aux_ref,
                      o_ref):
    cond = jnp.sum(wctx_ref[...] * lab_ref[0], axis=-1, keepdims=True)
    aux = aux_ref[0]
    cond = cond + aux[:, 0:1] + aux[:, 2:3] * aux[:, 1:2]

    out = jnp.dot(wx_ref[...], x_ref[0], preferred_element_type=jnp.float32)
    o_ref[0] = (out + cond).astype(o_ref.dtype)


def _pick_hw_tile(hw, max_tile=2048):
    if hw <= max_tile:
        return hw
    t = max_tile - (max_tile % 128)
    while t >= 128:
        if hw % t == 0:
            return t
        t -= 128
    return hw


def kernel(x, t, class_labels, w, bias, tproj):
    B, C, H, W = x.shape
    NC = class_labels.shape[1]
    C_out = w.shape[0]
    HW = H * W
    T_HW = _pick_hw_tile(HW)
    f32 = jnp.float32

    x3d = x.reshape(B, C, HW)
    wx = w[:, :C]
    wctx = w[:, C:]
    lab3 = class_labels.reshape(B, 1, NC).astype(f32)
    bias_b = jnp.broadcast_to(bias.reshape(1, C_out), (B, C_out))
    tproj_b = jnp.broadcast_to(tproj.reshape(1, C_out), (B, C_out))
    t_b = jnp.broadcast_to(t.astype(f32).reshape(B, 1), (B, C_out))
    aux = jnp.stack([bias_b, tproj_b, t_b], axis=-1)

    grid = (B, HW // T_HW)

    out3d = pl.pallas_call(
        _cond_conv_kernel,
        out_shape=jax.ShapeDtypeStruct((B, C_out, HW), x.dtype),
        grid=grid,
        in_specs=[
            pl.BlockSpec((1, C, T_HW), lambda b, h: (b, 0, h)),
            pl.BlockSpec((C_out, C), lambda b, h: (0, 0)),
            pl.BlockSpec((C_out, NC), lambda b, h: (0, 0)),
            pl.BlockSpec((1, 1, NC), lambda b, h: (b, 0, 0)),
            pl.BlockSpec((1, C_out, 3), lambda b, h: (b, 0, 0)),
        ],
        out_specs=pl.BlockSpec((1, C_out, T_HW), lambda b, h: (b, 0, h)),
        compiler_params=pltpu.CompilerParams(
            dimension_semantics=("parallel", "parallel")),
    )(x3d, wx, wctx, lab3, aux)

    return out3d.reshape(B, C_out, H, W)


# speedup vs baseline: 2.1938x; 1.0081x over previous
import jax
import jax.numpy as jnp
from jax.experimental import pallas as pl
from jax.experimental.pallas import tpu as pltpu


def _cond_conv_kernel(t_ref,
                      x_ref,
                      wx_ref,
                      wctx_ref,
                      lab_ref,
                      bias_ref,
                      tproj_ref,
                      o_ref):
    b = pl.program_id(0)
    cond = jnp.sum(wctx_ref[...] * lab_ref[0], axis=-1, keepdims=True)
    t_b = t_ref[b].astype(jnp.float32)
    cond = cond + bias_ref[...] + t_b * tproj_ref[...]

    out = jnp.dot(wx_ref[...], x_ref[0], preferred_element_type=jnp.float32)
    o_ref[0] = (out + cond).astype(o_ref.dtype)


def _pick_hw_tile(hw, max_tile=2048):
    if hw <= max_tile:
        return hw
    t = max_tile - (max_tile % 128)
    while t >= 128:
        if hw % t == 0:
            return t
        t -= 128
    return hw


def kernel(x, t, class_labels, w, bias, tproj):
    B, C, H, W = x.shape
    NC = class_labels.shape[1]
    C_out = w.shape[0]
    HW = H * W
    T_HW = _pick_hw_tile(HW)

    x3d = x.reshape(B, C, HW)
    lab3 = class_labels.reshape(B, 1, NC)

    grid = (B, HW // T_HW)

    out3d = pl.pallas_call(
        _cond_conv_kernel,
        out_shape=jax.ShapeDtypeStruct((B, C_out, HW), x.dtype),
        grid=grid,
        in_specs=[
            pl.BlockSpec(memory_space=pltpu.SMEM),
            pl.BlockSpec((1, C, T_HW), lambda b, h: (b, 0, h)),
            pl.BlockSpec((C_out, C), lambda b, h: (0, 0)),
            pl.BlockSpec((C_out, NC), lambda b, h: (0, 0)),
            pl.BlockSpec((1, 1, NC), lambda b, h: (b, 0, 0)),
            pl.BlockSpec((C_out, 1), lambda b, h: (0, 0)),
            pl.BlockSpec((C_out, 1), lambda b, h: (0, 0)),
        ],
        out_specs=pl.BlockSpec((1, C_out, T_HW), lambda b, h: (b, 0, h)),
        compiler_params=pltpu.CompilerParams(
            dimension_semantics=("parallel", "parallel")),
    )(t, x3d, w[:, :C], w[:, C:], lab3, bias, tproj)

    return out3d.reshape(B, C_out, H, W)


# speedup vs baseline: 2.1977x; 1.0018x over previous
import jax
import jax.numpy as jnp
from jax.experimental import pallas as pl
from jax.experimental.pallas import tpu as pltpu


def _cond_conv_kernel(t_ref,
                      x_ref,
                      wx_ref,
                      wctx_ref,
                      lab_ref,
                      bias_ref,
                      tproj_ref,
                      o_ref):
    b = pl.program_id(0)
    cond = jnp.sum(wctx_ref[...] * lab_ref[0], axis=-1, keepdims=True)
    t_b = t_ref[b].astype(jnp.float32)
    cond = cond + bias_ref[...] + t_b * tproj_ref[...]

    out = jnp.dot(wx_ref[...].astype(jnp.bfloat16),
                  x_ref[0].astype(jnp.bfloat16),
                  preferred_element_type=jnp.float32)
    o_ref[0] = (out + cond).astype(o_ref.dtype)


def _pick_hw_tile(hw, max_tile=2048):
    if hw <= max_tile:
        return hw
    t = max_tile - (max_tile % 128)
    while t >= 128:
        if hw % t == 0:
            return t
        t -= 128
    return hw


def kernel(x, t, class_labels, w, bias, tproj):
    B, C, H, W = x.shape
    NC = class_labels.shape[1]
    C_out = w.shape[0]
    HW = H * W
    T_HW = _pick_hw_tile(HW)

    x3d = x.reshape(B, C, HW)
    lab3 = class_labels.reshape(B, 1, NC)

    grid = (B, HW // T_HW)

    out3d = pl.pallas_call(
        _cond_conv_kernel,
        out_shape=jax.ShapeDtypeStruct((B, C_out, HW), x.dtype),
        grid=grid,
        in_specs=[
            pl.BlockSpec(memory_space=pltpu.SMEM),
            pl.BlockSpec((1, C, T_HW), lambda b, h: (b, 0, h)),
            pl.BlockSpec((C_out, C), lambda b, h: (0, 0)),
            pl.BlockSpec((C_out, NC), lambda b, h: (0, 0)),
            pl.BlockSpec((1, 1, NC), lambda b, h: (b, 0, 0)),
            pl.BlockSpec((C_out, 1), lambda b, h: (0, 0)),
            pl.BlockSpec((C_out, 1), lambda b, h: (0, 0)),
        ],
        out_specs=pl.BlockSpec((1, C_out, T_HW), lambda b, h: (b, 0, h)),
        compiler_params=pltpu.CompilerParams(
            dimension_semantics=("parallel", "parallel")),
    )(t, x3d, w[:, :C], w[:, C:], lab3, bias, tproj)

    return out3d.reshape(B, C_out, H, W)


# speedup vs baseline: 2.5538x; 1.1621x over previous
import jax
import jax.numpy as jnp
from jax.experimental import pallas as pl
from jax.experimental.pallas import tpu as pltpu


def _make_kernel(BB, C, NC, C_out, HW):
    def _cond_conv_kernel(t_ref,
                          x_ref,
                          w_ref,
                          lab_ref,
                          bias_ref,
                          tproj_ref,
                          o_ref):
        g = pl.program_id(0)
        wx = w_ref[:, :C]
        wctx = w_ref[:, C:]
        for j in range(BB):
            b = g * BB + j
            lab = lab_ref[pl.ds(b, 1), :]
            cond = jnp.sum(wctx * lab, axis=-1, keepdims=True)
            t_b = t_ref[b].astype(jnp.float32)
            cond = cond + bias_ref[...] + t_b * tproj_ref[...]
            out = jnp.dot(wx.astype(jnp.bfloat16),
                          x_ref[j].astype(jnp.bfloat16),
                          preferred_element_type=jnp.float32)
            o_ref[j] = (out + cond).astype(o_ref.dtype)
    return _cond_conv_kernel


def kernel(x, t, class_labels, w, bias, tproj):
    B, C, H, W = x.shape
    NC = class_labels.shape[1]
    C_out = w.shape[0]
    HW = H * W
    BB = 2 if B % 2 == 0 else 1

    x3d = x.reshape(B, C, HW)
    grid = (B // BB,)

    out3d = pl.pallas_call(
        _make_kernel(BB, C, NC, C_out, HW),
        out_shape=jax.ShapeDtypeStruct((B, C_out, HW), x.dtype),
        grid=grid,
        in_specs=[
            pl.BlockSpec(memory_space=pltpu.SMEM),
            pl.BlockSpec((BB, C, HW), lambda g: (g, 0, 0)),
            pl.BlockSpec((C_out, C + NC), lambda g: (0, 0)),
            pl.BlockSpec((B, NC), lambda g: (0, 0)),
            pl.BlockSpec((C_out, 1), lambda g: (0, 0)),
            pl.BlockSpec((C_out, 1), lambda g: (0, 0)),
        ],
        out_specs=pl.BlockSpec((BB, C_out, HW), lambda g: (g, 0, 0)),
        compiler_params=pltpu.CompilerParams(
            dimension_semantics=("parallel",)),
    )(t, x3d, w, class_labels, bias, tproj)

    return out3d.reshape(B, C_out, H, W)
